# Initial kernel scaffold; baseline (speedup 1.0000x reference)
import functools

import jax
import jax.numpy as jnp
from jax import lax
from jax.experimental import pallas as pl
from jax.experimental.pallas import tpu as pltpu
from jax.experimental.pallas import tpu_sc as plsc

N = 10000
D = 128
DE = 16
G = 16
NP = 10112
DUMMY = 10048
NC = 2
NS = 16
NTILES = NC * NS
E = 320000
EPT = 10240
CH = 128
NCH = EPT // CH
EPAD = EPT * NTILES
ROWS_PT = NP // NS
QW = 2 * DE

_mesh = plsc.VectorSubcoreMesh(core_axis_name="c", subcore_axis_name="s")


@functools.partial(
    pl.kernel,
    out_type=jax.ShapeDtypeStruct((NTILES, NP), jnp.float32),
    mesh=_mesh,
    scratch_types=[
        pltpu.VMEM((EPT,), jnp.int32),
        pltpu.VMEM((NP,), jnp.float32),
    ],
)
def _deg_kernel(row_hbm, out_hbm, rowbuf, hist):
    c = lax.axis_index("c")
    s = lax.axis_index("s")
    wid = s * NC + c
    pltpu.sync_copy(row_hbm.at[wid], rowbuf)
    zeros = jnp.zeros((16,), jnp.float32)

    def zbody(i, carry):
        hist[pl.ds(i * 16, 16)] = zeros
        return carry

    lax.fori_loop(0, NP // 16, zbody, 0)
    ones = jnp.ones((16,), jnp.float32)

    def body(i, carry):
        rv = rowbuf[pl.ds(i * 16, 16)]
        plsc.addupdate_scatter(hist, [rv], ones)
        return carry

    lax.fori_loop(0, EPT // 16, body, 0)
    pltpu.sync_copy(hist, out_hbm.at[wid])


def _make_scatter(with_stats):
    out_type = [jax.ShapeDtypeStruct((NC, NP, D), jnp.float32)]
    scratch = [
        pltpu.VMEM((CH,), jnp.int32),
        pltpu.VMEM((CH,), jnp.int32),
        pltpu.VMEM((CH, D), jnp.float32),
        pltpu.VMEM_SHARED((NP, D), jnp.float32),
    ]
    if with_stats:
        out_type.append(jax.ShapeDtypeStruct((NC, NP, QW), jnp.float32))
        scratch += [
            pltpu.VMEM((NP,), jnp.float32),
            pltpu.VMEM((CH, DE), jnp.float32),
            pltpu.VMEM((CH, QW), jnp.float32),
            pltpu.VMEM_SHARED((NP, QW), jnp.float32),
        ]

    def body(*args):
        if with_stats:
            (row_hbm, col_hbm, ms_hbm, z_hbm, dinv_hbm, ea_hbm,
             agg_out, s_out,
             rowchunk, colchunk, gbuf, agg_sp,
             dinv_v, eabuf, qbuf, s_sp) = args
        else:
            (row_hbm, col_hbm, ms_hbm, z_hbm,
             agg_out,
             rowchunk, colchunk, gbuf, agg_sp) = args
        c = lax.axis_index("c")
        s = lax.axis_index("s")
        wid = s * NC + c
        r0 = s * ROWS_PT
        pltpu.sync_copy(z_hbm.at[pl.ds(r0, ROWS_PT)],
                        agg_sp.at[pl.ds(r0, ROWS_PT)])
        if with_stats:
            pltpu.sync_copy(z_hbm.at[pl.ds(r0, ROWS_PT), pl.ds(0, QW)],
                            s_sp.at[pl.ds(r0, ROWS_PT)])
            pltpu.sync_copy(dinv_hbm, dinv_v)
            pltpu.sync_copy(z_hbm.at[pl.ds(0, CH), pl.ds(0, QW)], qbuf)
        plsc.subcore_barrier()

        iota = lax.iota(jnp.int32, 16)

        def chunk(k, carry):
            pltpu.sync_copy(row_hbm.at[wid, k], rowchunk)
            pltpu.sync_copy(col_hbm.at[wid, k], colchunk)
            pltpu.sync_copy(ms_hbm.at[rowchunk], gbuf)
            if with_stats:
                pltpu.sync_copy(ea_hbm.at[wid, k], eabuf)
                for g in range(CH // 16):
                    ids = iota + (g * 16)
                    rv = rowchunk[pl.ds(g * 16, 16)]
                    dv = plsc.load_gather(dinv_v, [rv])
                    plsc.store_scatter(
                        qbuf, [ids, jnp.full((16,), DE, jnp.int32)], dv)
                    for f in range(DE):
                        fv = jnp.full((16,), f, jnp.int32)
                        ev = plsc.load_gather(eabuf, [ids, fv])
                        plsc.store_scatter(qbuf, [ids, fv], ev * dv)
                pltpu.sync_copy(qbuf, s_sp.at[colchunk], add=True)
            pltpu.sync_copy(gbuf, agg_sp.at[colchunk], add=True)
            return carry

        lax.fori_loop(0, NCH, chunk, 0)
        plsc.subcore_barrier()
        pltpu.sync_copy(agg_sp.at[pl.ds(r0, ROWS_PT)],
                        agg_out.at[c, pl.ds(r0, ROWS_PT)])
        if with_stats:
            pltpu.sync_copy(s_sp.at[pl.ds(r0, ROWS_PT)],
                            s_out.at[c, pl.ds(r0, ROWS_PT)])

    return pl.kernel(body, out_type=out_type, mesh=_mesh,
                     scratch_types=scratch)


_scatter_stats = _make_scatter(True)
_scatter_plain = _make_scatter(False)


def _tc_enc_body(degp_ref, x_ref, wenc_ref, benc_ref, w0_ref,
                 dinv_ref, ms0_ref):
    deg = jnp.sum(degp_ref[...], axis=0) + 1.0
    dinv = lax.rsqrt(deg)
    dinv_ref[...] = dinv[:, None]
    h0 = jnp.dot(x_ref[...], wenc_ref[...].T,
                 preferred_element_type=jnp.float32) + benc_ref[...]
    ms0 = jnp.dot(h0, w0_ref[...].T,
                  preferred_element_type=jnp.float32) * dinv[:N, None]
    ms0_ref[0:N, :] = ms0
    ms0_ref[N:NP, :] = jnp.zeros((NP - N, D), jnp.float32)


_tc_enc = pl.pallas_call(
    _tc_enc_body,
    out_shape=[
        jax.ShapeDtypeStruct((NP, 1), jnp.float32),
        jax.ShapeDtypeStruct((NP, D), jnp.float32),
    ],
)


def _tc_layer_body(dinv_ref, ms_ref, agg_ref, s32_ref, ew_ref, ebb_ref,
                   wn_ref, msn_ref, *, apply_relu):
    d = dinv_ref[...]
    d2 = d * d
    s16 = s32_ref[0, :, 0:DE] + s32_ref[1, :, 0:DE]
    ssum = s32_ref[0, :, DE:DE + 1] + s32_ref[1, :, DE:DE + 1]
    ew = ew_ref[...]
    t = (d * jnp.dot(s16, ew.T, preferred_element_type=jnp.float32)
         + d2 * ew[:, 0][None, :]
         + (d * ssum + d2) * ebb_ref[...])
    agg = agg_ref[0] + agg_ref[1]
    h = d * (agg + d * ms_ref[...]) + t
    if apply_relu:
        h = jnp.maximum(h, 0.0)
    msn = jnp.dot(h, wn_ref[...].T,
                  preferred_element_type=jnp.float32) * d
    msn_ref[0:N, :] = msn[0:N, :]
    msn_ref[N:NP, :] = jnp.zeros((NP - N, D), jnp.float32)


_tc_layer = pl.pallas_call(
    functools.partial(_tc_layer_body, apply_relu=True),
    out_shape=jax.ShapeDtypeStruct((NP, D), jnp.float32),
)


def _tc_final_body(dinv_ref, ms_ref, agg_ref, s32_ref, ew_ref, ebb_ref,
                   batch_ref, pw_ref, pb_ref, out_ref):
    d = dinv_ref[...]
    d2 = d * d
    s16 = s32_ref[0, :, 0:DE] + s32_ref[1, :, 0:DE]
    ssum = s32_ref[0, :, DE:DE + 1] + s32_ref[1, :, DE:DE + 1]
    ew = ew_ref[...]
    t = (d * jnp.dot(s16, ew.T, preferred_element_type=jnp.float32)
         + d2 * ew[:, 0][None, :]
         + (d * ssum + d2) * ebb_ref[...])
    agg = agg_ref[0] + agg_ref[1]
    h = d * (agg + d * ms_ref[...]) + t
    hn = h[0:N, :]
    gid = lax.broadcasted_iota(jnp.int32, (N, G), 1)
    oh = jnp.where(batch_ref[...] == gid, 1.0, 0.0)
    sums = lax.dot_general(oh, hn, (((0,), (0,)), ((), ())),
                           preferred_element_type=jnp.float32)
    counts = jnp.sum(oh, axis=0)[:, None]
    pooled = sums / jnp.maximum(counts, 1.0)
    out_ref[...] = (jnp.dot(pooled, pw_ref[...].T,
                            preferred_element_type=jnp.float32)
                    + pb_ref[...])


_tc_final = pl.pallas_call(
    _tc_final_body,
    out_shape=jax.ShapeDtypeStruct((G, 1), jnp.float32),
)


def kernel(x, edge_index, edge_attr, batch, node_enc_w, node_enc_b,
           lin_w, lin_b, edge_w, edge_b, pred_w, pred_b):
    row = edge_index[0]
    col = edge_index[1]
    pad = EPAD - E
    padv = jnp.full((pad,), DUMMY, jnp.int32)
    rowp = jnp.concatenate([row, padv]).reshape(NTILES, NCH, CH)
    colp = jnp.concatenate([col, padv]).reshape(NTILES, NCH, CH)
    eap = jnp.concatenate(
        [edge_attr, jnp.zeros((pad, DE), jnp.float32)]
    ).reshape(NTILES, NCH, CH, DE)
    zeros_nd = jnp.zeros((NP, D), jnp.float32)

    degp = _deg_kernel(rowp.reshape(NTILES, EPT))
    dinv2, ms0 = _tc_enc(degp, x, node_enc_w, node_enc_b[None, :],
                         lin_w[0])
    dinv1 = dinv2.reshape(NP)

    agg0, s32 = _scatter_stats(rowp, colp, ms0, zeros_nd, dinv1, eap)
    s32s = jnp.stack([s32[0], s32[1]])

    ms = ms0
    agg = agg0
    for l in range(2):
        ms = _tc_layer(dinv2, ms, agg, s32s, edge_w[l],
                       (edge_b[l] + lin_b[l])[None, :], lin_w[l + 1])
        agg = _scatter_plain(rowp, colp, ms, zeros_nd)

    out = _tc_final(dinv2, ms, agg, s32s, edge_w[2],
                    (edge_b[2] + lin_b[2])[None, :],
                    batch[:, None], pred_w, pred_b[None, :])
    return out


# speedup vs baseline: 4.9731x; 4.9731x over previous
import functools

import jax
import jax.numpy as jnp
from jax import lax
from jax.experimental import pallas as pl
from jax.experimental.pallas import tpu as pltpu
from jax.experimental.pallas import tpu_sc as plsc

N = 10000
D = 128
DE = 16
G = 16
NP = 10112
DUMMY = 10048
NC = 2
NS = 16
NTILES = NC * NS
E = 320000
EPT = 10240
CH = 128
NCH = EPT // CH
EPAD = EPT * NTILES
ROWS_PT = NP // NS
QW = 2 * DE

@functools.cache
def _mesh():
    return plsc.VectorSubcoreMesh(core_axis_name="c", subcore_axis_name="s",
                                  num_cores=NC, num_subcores=NS)


_SC_PARAMS = pltpu.CompilerParams(needs_layout_passes=False)


def _deg_body(row_hbm, out_hbm, rowbuf, hist):
    c = lax.axis_index("c")
    s = lax.axis_index("s")
    wid = s * NC + c
    pltpu.sync_copy(row_hbm.at[wid], rowbuf)
    zeros = jnp.zeros((16,), jnp.float32)

    def zbody(i, carry):
        hist[pl.ds(i * 16, 16)] = zeros
        return carry

    lax.fori_loop(0, NP // 16, zbody, 0)
    ones = jnp.ones((16,), jnp.float32)

    def body(i, carry):
        rv = rowbuf[pl.ds(i * 16, 16)]
        plsc.addupdate_scatter(hist, [rv], ones)
        return carry

    lax.fori_loop(0, EPT // 16, body, 0)
    pltpu.sync_copy(hist, out_hbm.at[wid])


@functools.cache
def _deg_kernel():
    return pl.kernel(
        _deg_body,
        out_type=jax.ShapeDtypeStruct((NTILES, NP), jnp.float32),
        mesh=_mesh(),
        compiler_params=_SC_PARAMS,
        scratch_types=[
            pltpu.VMEM((EPT,), jnp.int32),
            pltpu.VMEM((NP,), jnp.float32),
        ],
    )


def _agg_body(row_hbm, col_hbm, ms_hbm, z_hbm, agg_out,
              rowchunk, colchunk, gbuf, agg_sp):
    c = lax.axis_index("c")
    s = lax.axis_index("s")
    wid = s * NC + c
    r0 = s * ROWS_PT
    pltpu.sync_copy(z_hbm.at[pl.ds(r0, ROWS_PT)],
                    agg_sp.at[pl.ds(r0, ROWS_PT)])
    plsc.subcore_barrier()

    def chunk(k, carry):
        pltpu.sync_copy(row_hbm.at[wid, k], rowchunk)
        pltpu.sync_copy(col_hbm.at[wid, k], colchunk)
        pltpu.sync_copy(ms_hbm.at[rowchunk], gbuf)
        pltpu.sync_copy(gbuf, agg_sp.at[colchunk], add=True)
        return carry

    lax.fori_loop(0, NCH, chunk, 0)
    plsc.subcore_barrier()
    pltpu.sync_copy(agg_sp.at[pl.ds(r0, ROWS_PT)],
                    agg_out.at[c, pl.ds(r0, ROWS_PT)])


@functools.cache
def _agg_kernel():
    return pl.kernel(
        _agg_body,
        out_type=jax.ShapeDtypeStruct((NC, NP, D), jnp.float32),
        mesh=_mesh(),
        compiler_params=_SC_PARAMS,
        scratch_types=[
            pltpu.VMEM((CH,), jnp.int32),
            pltpu.VMEM((CH,), jnp.int32),
            pltpu.VMEM((CH, D), jnp.float32),
            pltpu.VMEM_SHARED((NP, D), jnp.float32),
        ],
    )


def _wg_body(row_hbm, dinv_hbm, w_out, rowbuf, dinv_v, wbuf):
    c = lax.axis_index("c")
    s = lax.axis_index("s")
    wid = s * NC + c
    pltpu.sync_copy(row_hbm.at[wid], rowbuf)
    pltpu.sync_copy(dinv_hbm, dinv_v)

    def body(i, carry):
        rv = rowbuf[pl.ds(i * 16, 16)]
        wbuf[pl.ds(i * 16, 16)] = plsc.load_gather(dinv_v, [rv])
        return carry

    lax.fori_loop(0, EPT // 16, body, 0)
    pltpu.sync_copy(wbuf, w_out.at[wid])


@functools.cache
def _wg_kernel():
    return pl.kernel(
        _wg_body,
        out_type=jax.ShapeDtypeStruct((NTILES, EPT), jnp.float32),
        mesh=_mesh(),
        compiler_params=_SC_PARAMS,
        scratch_types=[
            pltpu.VMEM((EPT,), jnp.int32),
            pltpu.VMEM((NP,), jnp.float32),
            pltpu.VMEM((EPT,), jnp.float32),
        ],
    )


def _tc_q_body(w_ref, ea_ref, q_ref):
    w = w_ref[...]
    q_ref[:, 0:DE] = w * ea_ref[...]
    q_ref[:, DE:DE + 1] = w
    q_ref[:, DE + 1:QW] = jnp.zeros((w.shape[0], QW - DE - 1), jnp.float32)


_BQ = EPAD // 40
_tc_q = pl.pallas_call(
    _tc_q_body,
    grid=(40,),
    in_specs=[
        pl.BlockSpec((_BQ, 1), lambda i: (i, 0)),
        pl.BlockSpec((_BQ, DE), lambda i: (i, 0)),
    ],
    out_specs=pl.BlockSpec((_BQ, QW), lambda i: (i, 0)),
    out_shape=jax.ShapeDtypeStruct((EPAD, QW), jnp.float32),
)


def _qs_body(col_hbm, q_hbm, zq_hbm, s_out, colchunk, qchunk, s_sp):
    c = lax.axis_index("c")
    s = lax.axis_index("s")
    wid = s * NC + c
    r0 = s * ROWS_PT
    pltpu.sync_copy(zq_hbm.at[pl.ds(r0, ROWS_PT)],
                    s_sp.at[pl.ds(r0, ROWS_PT)])
    plsc.subcore_barrier()

    def chunk(k, carry):
        pltpu.sync_copy(col_hbm.at[wid, k], colchunk)
        pltpu.sync_copy(q_hbm.at[wid, k], qchunk)
        pltpu.sync_copy(qchunk, s_sp.at[colchunk], add=True)
        return carry

    lax.fori_loop(0, NCH, chunk, 0)
    plsc.subcore_barrier()
    pltpu.sync_copy(s_sp.at[pl.ds(r0, ROWS_PT)],
                    s_out.at[c, pl.ds(r0, ROWS_PT)])


@functools.cache
def _qs_kernel():
    return pl.kernel(
        _qs_body,
        out_type=jax.ShapeDtypeStruct((NC, NP, QW), jnp.float32),
        mesh=_mesh(),
        compiler_params=pltpu.CompilerParams(needs_layout_passes=False,
                                             use_tc_tiling_on_sc=False),
        scratch_types=[
            pltpu.VMEM((CH,), jnp.int32),
            pltpu.VMEM((CH, QW), jnp.float32),
            pltpu.VMEM_SHARED((NP, QW), jnp.float32),
        ],
    )


def _tc_enc_body(degp_ref, x_ref, wenc_ref, benc_ref, w0_ref,
                 dinv_ref, ms0_ref):
    deg = jnp.sum(degp_ref[...], axis=0) + 1.0
    dinv = 1.0 / jnp.sqrt(deg)
    dinv_ref[...] = dinv[:, None]
    h0 = jnp.dot(x_ref[...], wenc_ref[...].T,
                 preferred_element_type=jnp.float32) + benc_ref[...]
    ms0 = jnp.dot(h0, w0_ref[...].T,
                  preferred_element_type=jnp.float32) * dinv[:N, None]
    ms0_ref[0:N, :] = ms0
    ms0_ref[N:NP, :] = jnp.zeros((NP - N, D), jnp.float32)


_tc_enc = pl.pallas_call(
    _tc_enc_body,
    out_shape=[
        jax.ShapeDtypeStruct((NP, 1), jnp.float32),
        jax.ShapeDtypeStruct((NP, D), jnp.float32),
    ],
)


def _tc_layer_body(dinv_ref, ms_ref, agg_ref, s32_ref, ew_ref, ebb_ref,
                   wn_ref, msn_ref, *, apply_relu):
    d = dinv_ref[...]
    d2 = d * d
    s16 = s32_ref[0, :, 0:DE] + s32_ref[1, :, 0:DE]
    ssum = s32_ref[0, :, DE:DE + 1] + s32_ref[1, :, DE:DE + 1]
    ew = ew_ref[...]
    t = (d * jnp.dot(s16, ew.T, preferred_element_type=jnp.float32)
         + d2 * ew[:, 0][None, :]
         + (d * ssum + d2) * ebb_ref[...])
    agg = agg_ref[0] + agg_ref[1]
    h = d * (agg + ms_ref[...]) + t
    if apply_relu:
        h = jnp.maximum(h, 0.0)
    msn_ref[...] = jnp.dot(h, wn_ref[...].T,
                           preferred_element_type=jnp.float32) * d


_BN = NP // 8
_tc_layer = pl.pallas_call(
    functools.partial(_tc_layer_body, apply_relu=True),
    grid=(8,),
    in_specs=[
        pl.BlockSpec((_BN, 1), lambda i: (i, 0)),
        pl.BlockSpec((_BN, D), lambda i: (i, 0)),
        pl.BlockSpec((NC, _BN, D), lambda i: (0, i, 0)),
        pl.BlockSpec((NC, _BN, QW), lambda i: (0, i, 0)),
        pl.BlockSpec((D, DE), lambda i: (0, 0)),
        pl.BlockSpec((1, D), lambda i: (0, 0)),
        pl.BlockSpec((D, D), lambda i: (0, 0)),
    ],
    out_specs=pl.BlockSpec((_BN, D), lambda i: (i, 0)),
    out_shape=jax.ShapeDtypeStruct((NP, D), jnp.float32),
)


def _tc_final_body(dinv_ref, ms_ref, agg_ref, s32_ref, ew_ref, ebb_ref,
                   batch_ref, pw_ref, pb_ref, out_ref, sums_ref, counts_ref):
    i = pl.program_id(0)
    d = dinv_ref[...]
    d2 = d * d
    s16 = s32_ref[0, :, 0:DE] + s32_ref[1, :, 0:DE]
    ssum = s32_ref[0, :, DE:DE + 1] + s32_ref[1, :, DE:DE + 1]
    ew = ew_ref[...]
    t = (d * jnp.dot(s16, ew.T, preferred_element_type=jnp.float32)
         + d2 * ew[:, 0][None, :]
         + (d * ssum + d2) * ebb_ref[...])
    agg = agg_ref[0] + agg_ref[1]
    h = d * (agg + ms_ref[...]) + t
    gid = lax.broadcasted_iota(jnp.int32, (_BN, G), 1)
    oh = jnp.where(batch_ref[...] == gid, 1.0, 0.0)
    part = lax.dot_general(oh, h, (((0,), (0,)), ((), ())),
                           preferred_element_type=jnp.float32)
    pc = jnp.sum(oh, axis=0)[:, None]

    @pl.when(i == 0)
    def _():
        sums_ref[...] = jnp.zeros((G, D), jnp.float32)
        counts_ref[...] = jnp.zeros((G, 1), jnp.float32)

    sums_ref[...] += part
    counts_ref[...] += pc

    @pl.when(i == 7)
    def _():
        pooled = sums_ref[...] / jnp.maximum(counts_ref[...], 1.0)
        out_ref[...] = (jnp.sum(pooled * pw_ref[...], axis=1, keepdims=True)
                        + pb_ref[0, 0])


_tc_final = pl.pallas_call(
    _tc_final_body,
    grid=(8,),
    in_specs=[
        pl.BlockSpec((_BN, 1), lambda i: (i, 0)),
        pl.BlockSpec((_BN, D), lambda i: (i, 0)),
        pl.BlockSpec((NC, _BN, D), lambda i: (0, i, 0)),
        pl.BlockSpec((NC, _BN, QW), lambda i: (0, i, 0)),
        pl.BlockSpec((D, DE), lambda i: (0, 0)),
        pl.BlockSpec((1, D), lambda i: (0, 0)),
        pl.BlockSpec((_BN, 1), lambda i: (i, 0)),
        pl.BlockSpec((1, D), lambda i: (0, 0)),
        pl.BlockSpec((1, 1), lambda i: (0, 0)),
    ],
    out_specs=pl.BlockSpec((G, 1), lambda i: (0, 0)),
    out_shape=jax.ShapeDtypeStruct((G, 1), jnp.float32),
    scratch_shapes=[
        pltpu.VMEM((G, D), jnp.float32),
        pltpu.VMEM((G, 1), jnp.float32),
    ],
)


def kernel(x, edge_index, edge_attr, batch, node_enc_w, node_enc_b,
           lin_w, lin_b, edge_w, edge_b, pred_w, pred_b):
    row = edge_index[0]
    col = edge_index[1]
    pad = EPAD - E
    padv = jnp.full((pad,), DUMMY, jnp.int32)
    rowp = jnp.concatenate([row, padv]).reshape(NTILES, NCH, CH)
    colp = jnp.concatenate([col, padv]).reshape(NTILES, NCH, CH)
    ea_pad = jnp.concatenate(
        [edge_attr, jnp.zeros((pad, DE), jnp.float32)])
    zeros_nd = jnp.zeros((NP, D), jnp.float32)

    degp = _deg_kernel()(rowp.reshape(NTILES, EPT))
    dinv2, ms0 = _tc_enc(degp, x, node_enc_w, node_enc_b[None, :],
                         lin_w[0])
    dinv1 = dinv2.reshape(NP)

    zeros_nq = jnp.zeros((NP, QW), jnp.float32)
    w = _wg_kernel()(rowp.reshape(NTILES, EPT), dinv1)
    q = _tc_q(w.reshape(EPAD, 1), ea_pad)
    s32 = _qs_kernel()(colp, q.reshape(NTILES, NCH, CH, QW), zeros_nq)
    agg = _agg_kernel()(rowp, colp, ms0, zeros_nd)

    ms = ms0
    for l in range(2):
        ms = _tc_layer(dinv2, ms, agg, s32, edge_w[l],
                       (edge_b[l] + lin_b[l])[None, :], lin_w[l + 1])
        agg = _agg_kernel()(rowp, colp, ms, zeros_nd)

    batch_p = jnp.concatenate(
        [batch, jnp.full((NP - N,), G, jnp.int32)])[:, None]
    out = _tc_final(dinv2, ms, agg, s32, edge_w[2],
                    (edge_b[2] + lin_b[2])[None, :],
                    batch_p, pred_w, pred_b[:, None])
    return out


# speedup vs baseline: 7.7769x; 1.5638x over previous
import functools

import jax
import jax.numpy as jnp
from jax import lax
from jax.experimental import pallas as pl
from jax.experimental.pallas import tpu as pltpu
from jax.experimental.pallas import tpu_sc as plsc

N = 10000
D = 128
DE = 16
G = 16
NP = 10112
DUMMY = 10048
NC = 2
NS = 16
NTILES = NC * NS
E = 320000
EPT = 10240
CH = 128
NCH = EPT // CH
EPAD = EPT * NTILES
ROWS_PT = NP // NS
QW = 2 * DE

@functools.cache
def _mesh():
    return plsc.VectorSubcoreMesh(core_axis_name="c", subcore_axis_name="s",
                                  num_cores=NC, num_subcores=NS)


_SC_PARAMS = pltpu.CompilerParams(needs_layout_passes=False)


def _deg_body(row_hbm, out_hbm, rowbuf, hist):
    c = lax.axis_index("c")
    s = lax.axis_index("s")
    wid = s * NC + c
    pltpu.sync_copy(row_hbm.at[wid], rowbuf)
    zeros = jnp.zeros((16,), jnp.float32)

    def zbody(i, carry):
        hist[pl.ds(i * 16, 16)] = zeros
        return carry

    lax.fori_loop(0, NP // 16, zbody, 0)
    ones = jnp.ones((16,), jnp.float32)

    def body(i, carry):
        rv = rowbuf[pl.ds(i * 16, 16)]
        plsc.addupdate_scatter(hist, [rv], ones)
        return carry

    lax.fori_loop(0, EPT // 16, body, 0)
    pltpu.sync_copy(hist, out_hbm.at[wid])


@functools.cache
def _deg_kernel():
    return pl.kernel(
        _deg_body,
        out_type=jax.ShapeDtypeStruct((NTILES, NP), jnp.float32),
        mesh=_mesh(),
        compiler_params=_SC_PARAMS,
        scratch_types=[
            pltpu.VMEM((EPT,), jnp.int32),
            pltpu.VMEM((NP,), jnp.float32),
        ],
    )


DH = D // NC
EPT2 = EPAD // NS
NCH2 = EPT2 // CH


def _agg_body(row_hbm, col_hbm, ms_hbm, z_hbm, agg_out,
              row2d, col2d, gbuf0, gbuf1, agg_sp,
              sg0, sg1, ss0, ss1):
    c = lax.axis_index("c")
    s = lax.axis_index("s")
    r0 = s * ROWS_PT
    pltpu.sync_copy(z_hbm.at[pl.ds(r0, ROWS_PT)],
                    agg_sp.at[pl.ds(r0, ROWS_PT)])
    pltpu.sync_copy(row_hbm.at[c, s], row2d)
    pltpu.sync_copy(col_hbm.at[s], col2d)
    plsc.subcore_barrier()

    gbufs = (gbuf0, gbuf1)
    sgs = (sg0, sg1)
    sss = (ss0, ss1)

    def gather(k, b):
        pltpu.async_copy(ms_hbm.at[row2d.at[k]], gbufs[b], sgs[b])

    def wait_gather(k, b):
        pltpu.make_async_copy(ms_hbm.at[row2d.at[k]], gbufs[b],
                              sgs[b]).wait()

    def scatter(k, b):
        pltpu.async_copy(gbufs[b], agg_sp.at[col2d.at[k]], sss[b],
                         add=True)

    def wait_scatter(k, b):
        pltpu.make_async_copy(gbufs[b], agg_sp.at[col2d.at[k]],
                              sss[b]).wait()

    def sub(k, b):
        @pl.when(k >= 2)
        def _():
            wait_scatter(k - 2, b)

        gather(k, b)

        @pl.when(k >= 1)
        def _():
            wait_gather(k - 1, 1 - b)
            scatter(k - 1, 1 - b)

    def chunk(i, carry):
        sub(2 * i, 0)
        sub(2 * i + 1, 1)
        return carry

    lax.fori_loop(0, NCH2 // 2, chunk, 0)
    wait_gather(NCH2 - 1, 1)
    scatter(NCH2 - 1, 1)
    wait_scatter(NCH2 - 2, 0)
    wait_scatter(NCH2 - 1, 1)
    plsc.subcore_barrier()
    pltpu.sync_copy(agg_sp.at[pl.ds(r0, ROWS_PT)],
                    agg_out.at[c, pl.ds(r0, ROWS_PT)])


@functools.cache
def _agg_kernel():
    return pl.kernel(
        _agg_body,
        out_type=jax.ShapeDtypeStruct((NC, NP, DH), jnp.float32),
        mesh=_mesh(),
        compiler_params=pltpu.CompilerParams(needs_layout_passes=False,
                                             use_tc_tiling_on_sc=False),
        scratch_types=[
            pltpu.VMEM((NCH2, CH), jnp.int32),
            pltpu.VMEM((NCH2, CH), jnp.int32),
            pltpu.VMEM((CH, DH), jnp.float32),
            pltpu.VMEM((CH, DH), jnp.float32),
            pltpu.VMEM_SHARED((NP, DH), jnp.float32),
            pltpu.SemaphoreType.DMA,
            pltpu.SemaphoreType.DMA,
            pltpu.SemaphoreType.DMA,
            pltpu.SemaphoreType.DMA,
        ],
    )


def _wg_body(row_hbm, dinv_hbm, w_out, rowbuf, dinv_v, wbuf):
    c = lax.axis_index("c")
    s = lax.axis_index("s")
    wid = s * NC + c
    pltpu.sync_copy(row_hbm.at[wid], rowbuf)
    pltpu.sync_copy(dinv_hbm, dinv_v)

    def body(i, carry):
        rv = rowbuf[pl.ds(i * 16, 16)]
        wbuf[pl.ds(i * 16, 16)] = plsc.load_gather(dinv_v, [rv])
        return carry

    lax.fori_loop(0, EPT // 16, body, 0)
    pltpu.sync_copy(wbuf, w_out.at[wid])


@functools.cache
def _wg_kernel():
    return pl.kernel(
        _wg_body,
        out_type=jax.ShapeDtypeStruct((NTILES, EPT), jnp.float32),
        mesh=_mesh(),
        compiler_params=_SC_PARAMS,
        scratch_types=[
            pltpu.VMEM((EPT,), jnp.int32),
            pltpu.VMEM((NP,), jnp.float32),
            pltpu.VMEM((EPT,), jnp.float32),
        ],
    )


def _tc_q_body(w_ref, ea_ref, q_ref):
    w = w_ref[...]
    q_ref[:, 0:DE] = w * ea_ref[...]
    q_ref[:, DE:DE + 1] = w
    q_ref[:, DE + 1:QW] = jnp.zeros((w.shape[0], QW - DE - 1), jnp.float32)


_BQ = EPAD // 40
_tc_q = pl.pallas_call(
    _tc_q_body,
    grid=(40,),
    in_specs=[
        pl.BlockSpec((_BQ, 1), lambda i: (i, 0)),
        pl.BlockSpec((_BQ, DE), lambda i: (i, 0)),
    ],
    out_specs=pl.BlockSpec((_BQ, QW), lambda i: (i, 0)),
    out_shape=jax.ShapeDtypeStruct((EPAD, QW), jnp.float32),
)


def _qs_body(col_hbm, q_hbm, zq_hbm, s_out, colchunk, qchunk, s_sp):
    c = lax.axis_index("c")
    s = lax.axis_index("s")
    wid = s * NC + c
    r0 = s * ROWS_PT
    pltpu.sync_copy(zq_hbm.at[pl.ds(r0, ROWS_PT)],
                    s_sp.at[pl.ds(r0, ROWS_PT)])
    plsc.subcore_barrier()

    def chunk(k, carry):
        pltpu.sync_copy(col_hbm.at[wid, k], colchunk)
        pltpu.sync_copy(q_hbm.at[wid, k], qchunk)
        pltpu.sync_copy(qchunk, s_sp.at[colchunk], add=True)
        return carry

    lax.fori_loop(0, NCH, chunk, 0)
    plsc.subcore_barrier()
    pltpu.sync_copy(s_sp.at[pl.ds(r0, ROWS_PT)],
                    s_out.at[c, pl.ds(r0, ROWS_PT)])


@functools.cache
def _qs_kernel():
    return pl.kernel(
        _qs_body,
        out_type=jax.ShapeDtypeStruct((NC, NP, QW), jnp.float32),
        mesh=_mesh(),
        compiler_params=pltpu.CompilerParams(needs_layout_passes=False,
                                             use_tc_tiling_on_sc=False),
        scratch_types=[
            pltpu.VMEM((CH,), jnp.int32),
            pltpu.VMEM((CH, QW), jnp.float32),
            pltpu.VMEM_SHARED((NP, QW), jnp.float32),
        ],
    )


def _tc_enc_body(degp_ref, x_ref, wenc_ref, benc_ref, w0_ref,
                 dinv_ref, ms0_ref):
    deg = jnp.sum(degp_ref[...], axis=0) + 1.0
    dinv = 1.0 / jnp.sqrt(deg)
    dinv_ref[...] = dinv[:, None]
    h0 = jnp.dot(x_ref[...], wenc_ref[...].T,
                 preferred_element_type=jnp.float32) + benc_ref[...]
    ms0 = jnp.dot(h0, w0_ref[...].T,
                  preferred_element_type=jnp.float32) * dinv[:N, None]
    zpad = jnp.zeros((NP - N, DH), jnp.float32)
    ms0_ref[0, 0:N, :] = ms0[:, 0:DH]
    ms0_ref[0, N:NP, :] = zpad
    ms0_ref[1, 0:N, :] = ms0[:, DH:D]
    ms0_ref[1, N:NP, :] = zpad


_tc_enc = pl.pallas_call(
    _tc_enc_body,
    out_shape=[
        jax.ShapeDtypeStruct((NP, 1), jnp.float32),
        jax.ShapeDtypeStruct((NC, NP, DH), jnp.float32),
    ],
)


def _tc_layer_body(dinv_ref, ms_ref, agg_ref, s32_ref, ew_ref, ebb_ref,
                   wn_ref, msn_ref, *, apply_relu):
    d = dinv_ref[...]
    d2 = d * d
    s16 = s32_ref[0, :, 0:DE] + s32_ref[1, :, 0:DE]
    ssum = s32_ref[0, :, DE:DE + 1] + s32_ref[1, :, DE:DE + 1]
    ew = ew_ref[...]
    t = (d * jnp.dot(s16, ew.T, preferred_element_type=jnp.float32)
         + d2 * ew[:, 0][None, :]
         + (d * ssum + d2) * ebb_ref[...])
    agg = jnp.concatenate([agg_ref[0], agg_ref[1]], axis=1)
    ms = jnp.concatenate([ms_ref[0], ms_ref[1]], axis=1)
    h = d * (agg + ms) + t
    if apply_relu:
        h = jnp.maximum(h, 0.0)
    msn = jnp.dot(h, wn_ref[...].T,
                  preferred_element_type=jnp.float32) * d
    msn_ref[0] = msn[:, 0:DH]
    msn_ref[1] = msn[:, DH:D]


_BN = NP // 8
_tc_layer = pl.pallas_call(
    functools.partial(_tc_layer_body, apply_relu=True),
    grid=(8,),
    in_specs=[
        pl.BlockSpec((_BN, 1), lambda i: (i, 0)),
        pl.BlockSpec((NC, _BN, DH), lambda i: (0, i, 0)),
        pl.BlockSpec((NC, _BN, DH), lambda i: (0, i, 0)),
        pl.BlockSpec((NC, _BN, QW), lambda i: (0, i, 0)),
        pl.BlockSpec((D, DE), lambda i: (0, 0)),
        pl.BlockSpec((1, D), lambda i: (0, 0)),
        pl.BlockSpec((D, D), lambda i: (0, 0)),
    ],
    out_specs=pl.BlockSpec((NC, _BN, DH), lambda i: (0, i, 0)),
    out_shape=jax.ShapeDtypeStruct((NC, NP, DH), jnp.float32),
)


def _tc_final_body(dinv_ref, ms_ref, agg_ref, s32_ref, ew_ref, ebb_ref,
                   batch_ref, pw_ref, pb_ref, out_ref, sums_ref, counts_ref):
    i = pl.program_id(0)
    d = dinv_ref[...]
    d2 = d * d
    s16 = s32_ref[0, :, 0:DE] + s32_ref[1, :, 0:DE]
    ssum = s32_ref[0, :, DE:DE + 1] + s32_ref[1, :, DE:DE + 1]
    ew = ew_ref[...]
    t = (d * jnp.dot(s16, ew.T, preferred_element_type=jnp.float32)
         + d2 * ew[:, 0][None, :]
         + (d * ssum + d2) * ebb_ref[...])
    agg = jnp.concatenate([agg_ref[0], agg_ref[1]], axis=1)
    ms = jnp.concatenate([ms_ref[0], ms_ref[1]], axis=1)
    h = d * (agg + ms) + t
    gid = lax.broadcasted_iota(jnp.int32, (_BN, G), 1)
    oh = jnp.where(batch_ref[...] == gid, 1.0, 0.0)
    part = lax.dot_general(oh, h, (((0,), (0,)), ((), ())),
                           preferred_element_type=jnp.float32)
    pc = jnp.sum(oh, axis=0)[:, None]

    @pl.when(i == 0)
    def _():
        sums_ref[...] = jnp.zeros((G, D), jnp.float32)
        counts_ref[...] = jnp.zeros((G, 1), jnp.float32)

    sums_ref[...] += part
    counts_ref[...] += pc

    @pl.when(i == 7)
    def _():
        pooled = sums_ref[...] / jnp.maximum(counts_ref[...], 1.0)
        out_ref[...] = (jnp.sum(pooled * pw_ref[...], axis=1, keepdims=True)
                        + pb_ref[0, 0])


_tc_final = pl.pallas_call(
    _tc_final_body,
    grid=(8,),
    in_specs=[
        pl.BlockSpec((_BN, 1), lambda i: (i, 0)),
        pl.BlockSpec((NC, _BN, DH), lambda i: (0, i, 0)),
        pl.BlockSpec((NC, _BN, DH), lambda i: (0, i, 0)),
        pl.BlockSpec((NC, _BN, QW), lambda i: (0, i, 0)),
        pl.BlockSpec((D, DE), lambda i: (0, 0)),
        pl.BlockSpec((1, D), lambda i: (0, 0)),
        pl.BlockSpec((_BN, 1), lambda i: (i, 0)),
        pl.BlockSpec((1, D), lambda i: (0, 0)),
        pl.BlockSpec((1, 1), lambda i: (0, 0)),
    ],
    out_specs=pl.BlockSpec((G, 1), lambda i: (0, 0)),
    out_shape=jax.ShapeDtypeStruct((G, 1), jnp.float32),
    scratch_shapes=[
        pltpu.VMEM((G, D), jnp.float32),
        pltpu.VMEM((G, 1), jnp.float32),
    ],
)


def kernel(x, edge_index, edge_attr, batch, node_enc_w, node_enc_b,
           lin_w, lin_b, edge_w, edge_b, pred_w, pred_b):
    row = edge_index[0]
    col = edge_index[1]
    pad = EPAD - E
    padv = jnp.full((pad,), DUMMY, jnp.int32)
    row_pad = jnp.concatenate([row, padv])
    col_pad = jnp.concatenate([col, padv])
    rowp = row_pad.reshape(NTILES, NCH, CH)
    colp = col_pad.reshape(NTILES, NCH, CH)
    r3 = row_pad.reshape(NS, NCH2, CH)
    row_sc = jnp.stack([r3, r3 + NP])
    col_sc = col_pad.reshape(NS, NCH2, CH)
    ea_pad = jnp.concatenate(
        [edge_attr, jnp.zeros((pad, DE), jnp.float32)])
    zeros_nh = jnp.zeros((NP, DH), jnp.float32)

    degp = _deg_kernel()(rowp.reshape(NTILES, EPT))
    dinv2, ms0 = _tc_enc(degp, x, node_enc_w, node_enc_b[None, :],
                         lin_w[0])
    dinv1 = dinv2.reshape(NP)

    zeros_nq = jnp.zeros((NP, QW), jnp.float32)
    w = _wg_kernel()(rowp.reshape(NTILES, EPT), dinv1)
    q = _tc_q(w.reshape(EPAD, 1), ea_pad)
    s32 = _qs_kernel()(colp, q.reshape(NTILES, NCH, CH, QW), zeros_nq)
    agg = _agg_kernel()(row_sc, col_sc, ms0.reshape(NC * NP, DH), zeros_nh)

    ms = ms0
    for l in range(2):
        ms = _tc_layer(dinv2, ms, agg, s32, edge_w[l],
                       (edge_b[l] + lin_b[l])[None, :], lin_w[l + 1])
        agg = _agg_kernel()(row_sc, col_sc, ms.reshape(NC * NP, DH),
                            zeros_nh)

    batch_p = jnp.concatenate(
        [batch, jnp.full((NP - N,), G, jnp.int32)])[:, None]
    out = _tc_final(dinv2, ms, agg, s32, edge_w[2],
                    (edge_b[2] + lin_b[2])[None, :],
                    batch_p, pred_w, pred_b[:, None])
    return out


# speedup vs baseline: 7.9501x; 1.0223x over previous
import functools

import jax
import jax.numpy as jnp
from jax import lax
from jax.experimental import pallas as pl
from jax.experimental.pallas import tpu as pltpu
from jax.experimental.pallas import tpu_sc as plsc

N = 10000
D = 128
DE = 16
G = 16
NP = 10112
DUMMY = 10048
NC = 2
NS = 16
NTILES = NC * NS
E = 320000
EPT = 10240
CH = 128
NCH = EPT // CH
EPAD = EPT * NTILES
ROWS_PT = NP // NS
QW = 2 * DE

@functools.cache
def _mesh():
    return plsc.VectorSubcoreMesh(core_axis_name="c", subcore_axis_name="s",
                                  num_cores=NC, num_subcores=NS)


_SC_PARAMS = pltpu.CompilerParams(needs_layout_passes=False)


def _deg_body(row_hbm, out_hbm, rowbuf, hist):
    c = lax.axis_index("c")
    s = lax.axis_index("s")
    wid = s * NC + c
    pltpu.sync_copy(row_hbm.at[wid], rowbuf)
    zeros = jnp.zeros((16,), jnp.float32)

    def zbody(i, carry):
        for j in range(8):
            hist[pl.ds(i * 128 + j * 16, 16)] = zeros
        return carry

    lax.fori_loop(0, NP // 128, zbody, 0)
    ones = jnp.ones((16,), jnp.float32)

    def body(i, carry):
        for j in range(8):
            rv = rowbuf[pl.ds(i * 128 + j * 16, 16)]
            plsc.addupdate_scatter(hist, [rv], ones)
        return carry

    lax.fori_loop(0, EPT // 128, body, 0)
    pltpu.sync_copy(hist, out_hbm.at[wid])


@functools.cache
def _deg_kernel():
    return pl.kernel(
        _deg_body,
        out_type=jax.ShapeDtypeStruct((NTILES, NP), jnp.float32),
        mesh=_mesh(),
        compiler_params=_SC_PARAMS,
        scratch_types=[
            pltpu.VMEM((EPT,), jnp.int32),
            pltpu.VMEM((NP,), jnp.float32),
        ],
    )


DH = D // NC
EPT2 = EPAD // NS
NCH2 = EPT2 // CH


def _agg_body(row_hbm, col_hbm, ms_hbm, z_hbm, agg_out,
              row2d, col2d, gbuf0, gbuf1, gbuf2, gbuf3, agg_sp,
              sg0, sg1, sg2, sg3, ss0, ss1, ss2, ss3):
    c = lax.axis_index("c")
    s = lax.axis_index("s")
    r0 = s * ROWS_PT
    pltpu.sync_copy(z_hbm.at[pl.ds(r0, ROWS_PT)],
                    agg_sp.at[pl.ds(r0, ROWS_PT)])
    pltpu.sync_copy(row_hbm.at[c, s], row2d)
    pltpu.sync_copy(col_hbm.at[s], col2d)
    plsc.subcore_barrier()

    gbufs = (gbuf0, gbuf1, gbuf2, gbuf3)
    sgs = (sg0, sg1, sg2, sg3)
    sss = (ss0, ss1, ss2, ss3)
    P = 4

    def gather(k, b):
        pltpu.async_copy(ms_hbm.at[row2d.at[k]], gbufs[b], sgs[b])

    def wait_gather(k, b):
        pltpu.make_async_copy(ms_hbm.at[row2d.at[k]], gbufs[b],
                              sgs[b]).wait()

    def scatter(k, b):
        pltpu.async_copy(gbufs[b], agg_sp.at[col2d.at[k]], sss[b],
                         add=True)

    def wait_scatter(k, b):
        pltpu.make_async_copy(gbufs[b], agg_sp.at[col2d.at[k]],
                              sss[b]).wait()

    def sub(k, b):
        @pl.when(k >= P)
        def _():
            wait_scatter(k - P, b)

        gather(k, b)

        @pl.when(k >= P - 1)
        def _():
            wait_gather(k - (P - 1), (b + 1) % P)
            scatter(k - (P - 1), (b + 1) % P)

    def chunk(i, carry):
        for j in range(P):
            sub(P * i + j, j)
        return carry

    lax.fori_loop(0, NCH2 // P, chunk, 0)
    for k in range(NCH2 - (P - 1), NCH2):
        wait_gather(k, k % P)
        scatter(k, k % P)
    for k in range(NCH2 - P, NCH2):
        wait_scatter(k, k % P)
    plsc.subcore_barrier()
    pltpu.sync_copy(agg_sp.at[pl.ds(r0, ROWS_PT)],
                    agg_out.at[c, pl.ds(r0, ROWS_PT)])


@functools.cache
def _agg_kernel():
    return pl.kernel(
        _agg_body,
        out_type=jax.ShapeDtypeStruct((NC, NP, DH), jnp.float32),
        mesh=_mesh(),
        compiler_params=pltpu.CompilerParams(needs_layout_passes=False,
                                             use_tc_tiling_on_sc=False),
        scratch_types=[
            pltpu.VMEM((NCH2, CH), jnp.int32),
            pltpu.VMEM((NCH2, CH), jnp.int32),
            pltpu.VMEM((CH, DH), jnp.float32),
            pltpu.VMEM((CH, DH), jnp.float32),
            pltpu.VMEM((CH, DH), jnp.float32),
            pltpu.VMEM((CH, DH), jnp.float32),
            pltpu.VMEM_SHARED((NP, DH), jnp.float32),
        ] + [pltpu.SemaphoreType.DMA] * 8,
    )


def _wg_body(row_hbm, dinv_hbm, w_out, rowbuf, dinv_v, wbuf):
    c = lax.axis_index("c")
    s = lax.axis_index("s")
    wid = s * NC + c
    pltpu.sync_copy(row_hbm.at[wid], rowbuf)
    pltpu.sync_copy(dinv_hbm, dinv_v)

    def body(i, carry):
        for j in range(8):
            rv = rowbuf[pl.ds(i * 128 + j * 16, 16)]
            wbuf[pl.ds(i * 128 + j * 16, 16)] = plsc.load_gather(
                dinv_v, [rv])
        return carry

    lax.fori_loop(0, EPT // 128, body, 0)
    pltpu.sync_copy(wbuf, w_out.at[wid])


@functools.cache
def _wg_kernel():
    return pl.kernel(
        _wg_body,
        out_type=jax.ShapeDtypeStruct((NTILES, EPT), jnp.float32),
        mesh=_mesh(),
        compiler_params=_SC_PARAMS,
        scratch_types=[
            pltpu.VMEM((EPT,), jnp.int32),
            pltpu.VMEM((NP,), jnp.float32),
            pltpu.VMEM((EPT,), jnp.float32),
        ],
    )


def _tc_q_body(w_ref, ea_ref, q_ref):
    w = w_ref[...]
    q_ref[:, 0:DE] = w * ea_ref[...]
    q_ref[:, DE:DE + 1] = w
    q_ref[:, DE + 1:QW] = jnp.zeros((w.shape[0], QW - DE - 1), jnp.float32)


_BQ = EPAD // 40
_tc_q = pl.pallas_call(
    _tc_q_body,
    grid=(40,),
    in_specs=[
        pl.BlockSpec((_BQ, 1), lambda i: (i, 0)),
        pl.BlockSpec((_BQ, DE), lambda i: (i, 0)),
    ],
    out_specs=pl.BlockSpec((_BQ, QW), lambda i: (i, 0)),
    out_shape=jax.ShapeDtypeStruct((EPAD, QW), jnp.float32),
)


def _qs_body(col_hbm, q_hbm, zq_hbm, s_out, colchunk, qchunk, s_sp):
    c = lax.axis_index("c")
    s = lax.axis_index("s")
    wid = s * NC + c
    r0 = s * ROWS_PT
    pltpu.sync_copy(zq_hbm.at[pl.ds(r0, ROWS_PT)],
                    s_sp.at[pl.ds(r0, ROWS_PT)])
    plsc.subcore_barrier()

    def chunk(k, carry):
        pltpu.sync_copy(col_hbm.at[wid, k], colchunk)
        pltpu.sync_copy(q_hbm.at[wid, k], qchunk)
        pltpu.sync_copy(qchunk, s_sp.at[colchunk], add=True)
        return carry

    lax.fori_loop(0, NCH, chunk, 0)
    plsc.subcore_barrier()
    pltpu.sync_copy(s_sp.at[pl.ds(r0, ROWS_PT)],
                    s_out.at[c, pl.ds(r0, ROWS_PT)])


@functools.cache
def _qs_kernel():
    return pl.kernel(
        _qs_body,
        out_type=jax.ShapeDtypeStruct((NC, NP, QW), jnp.float32),
        mesh=_mesh(),
        compiler_params=pltpu.CompilerParams(needs_layout_passes=False,
                                             use_tc_tiling_on_sc=False),
        scratch_types=[
            pltpu.VMEM((CH,), jnp.int32),
            pltpu.VMEM((CH, QW), jnp.float32),
            pltpu.VMEM_SHARED((NP, QW), jnp.float32),
        ],
    )


def _tc_enc_body(degp_ref, x_ref, wenc_ref, benc_ref, w0_ref,
                 dinv_ref, ms0_ref):
    deg = jnp.sum(degp_ref[...], axis=0) + 1.0
    dinv = 1.0 / jnp.sqrt(deg)
    dinv_ref[...] = dinv[:, None]
    h0 = jnp.dot(x_ref[...], wenc_ref[...].T,
                 preferred_element_type=jnp.float32) + benc_ref[...]
    ms0 = jnp.dot(h0, w0_ref[...].T,
                  preferred_element_type=jnp.float32) * dinv[:N, None]
    zpad = jnp.zeros((NP - N, DH), jnp.float32)
    ms0_ref[0, 0:N, :] = ms0[:, 0:DH]
    ms0_ref[0, N:NP, :] = zpad
    ms0_ref[1, 0:N, :] = ms0[:, DH:D]
    ms0_ref[1, N:NP, :] = zpad


_tc_enc = pl.pallas_call(
    _tc_enc_body,
    out_shape=[
        jax.ShapeDtypeStruct((NP, 1), jnp.float32),
        jax.ShapeDtypeStruct((NC, NP, DH), jnp.float32),
    ],
)


def _tc_layer_body(dinv_ref, ms_ref, agg_ref, s32_ref, ew_ref, ebb_ref,
                   wn_ref, msn_ref, *, apply_relu):
    d = dinv_ref[...]
    d2 = d * d
    s16 = s32_ref[0, :, 0:DE] + s32_ref[1, :, 0:DE]
    ssum = s32_ref[0, :, DE:DE + 1] + s32_ref[1, :, DE:DE + 1]
    ew = ew_ref[...]
    t = (d * jnp.dot(s16, ew.T, preferred_element_type=jnp.float32)
         + d2 * ew[:, 0][None, :]
         + (d * ssum + d2) * ebb_ref[...])
    agg = jnp.concatenate([agg_ref[0], agg_ref[1]], axis=1)
    ms = jnp.concatenate([ms_ref[0], ms_ref[1]], axis=1)
    h = d * (agg + ms) + t
    if apply_relu:
        h = jnp.maximum(h, 0.0)
    msn = jnp.dot(h, wn_ref[...].T,
                  preferred_element_type=jnp.float32) * d
    msn_ref[0] = msn[:, 0:DH]
    msn_ref[1] = msn[:, DH:D]


_BN = NP // 8
_tc_layer = pl.pallas_call(
    functools.partial(_tc_layer_body, apply_relu=True),
    grid=(8,),
    in_specs=[
        pl.BlockSpec((_BN, 1), lambda i: (i, 0)),
        pl.BlockSpec((NC, _BN, DH), lambda i: (0, i, 0)),
        pl.BlockSpec((NC, _BN, DH), lambda i: (0, i, 0)),
        pl.BlockSpec((NC, _BN, QW), lambda i: (0, i, 0)),
        pl.BlockSpec((D, DE), lambda i: (0, 0)),
        pl.BlockSpec((1, D), lambda i: (0, 0)),
        pl.BlockSpec((D, D), lambda i: (0, 0)),
    ],
    out_specs=pl.BlockSpec((NC, _BN, DH), lambda i: (0, i, 0)),
    out_shape=jax.ShapeDtypeStruct((NC, NP, DH), jnp.float32),
)


def _tc_final_body(dinv_ref, ms_ref, agg_ref, s32_ref, ew_ref, ebb_ref,
                   batch_ref, pw_ref, pb_ref, out_ref, sums_ref, counts_ref):
    i = pl.program_id(0)
    d = dinv_ref[...]
    d2 = d * d
    s16 = s32_ref[0, :, 0:DE] + s32_ref[1, :, 0:DE]
    ssum = s32_ref[0, :, DE:DE + 1] + s32_ref[1, :, DE:DE + 1]
    ew = ew_ref[...]
    t = (d * jnp.dot(s16, ew.T, preferred_element_type=jnp.float32)
         + d2 * ew[:, 0][None, :]
         + (d * ssum + d2) * ebb_ref[...])
    agg = jnp.concatenate([agg_ref[0], agg_ref[1]], axis=1)
    ms = jnp.concatenate([ms_ref[0], ms_ref[1]], axis=1)
    h = d * (agg + ms) + t
    gid = lax.broadcasted_iota(jnp.int32, (_BN, G), 1)
    oh = jnp.where(batch_ref[...] == gid, 1.0, 0.0)
    part = lax.dot_general(oh, h, (((0,), (0,)), ((), ())),
                           preferred_element_type=jnp.float32)
    pc = jnp.sum(oh, axis=0)[:, None]

    @pl.when(i == 0)
    def _():
        sums_ref[...] = jnp.zeros((G, D), jnp.float32)
        counts_ref[...] = jnp.zeros((G, 1), jnp.float32)

    sums_ref[...] += part
    counts_ref[...] += pc

    @pl.when(i == 7)
    def _():
        pooled = sums_ref[...] / jnp.maximum(counts_ref[...], 1.0)
        out_ref[...] = (jnp.sum(pooled * pw_ref[...], axis=1, keepdims=True)
                        + pb_ref[0, 0])


_tc_final = pl.pallas_call(
    _tc_final_body,
    grid=(8,),
    in_specs=[
        pl.BlockSpec((_BN, 1), lambda i: (i, 0)),
        pl.BlockSpec((NC, _BN, DH), lambda i: (0, i, 0)),
        pl.BlockSpec((NC, _BN, DH), lambda i: (0, i, 0)),
        pl.BlockSpec((NC, _BN, QW), lambda i: (0, i, 0)),
        pl.BlockSpec((D, DE), lambda i: (0, 0)),
        pl.BlockSpec((1, D), lambda i: (0, 0)),
        pl.BlockSpec((_BN, 1), lambda i: (i, 0)),
        pl.BlockSpec((1, D), lambda i: (0, 0)),
        pl.BlockSpec((1, 1), lambda i: (0, 0)),
    ],
    out_specs=pl.BlockSpec((G, 1), lambda i: (0, 0)),
    out_shape=jax.ShapeDtypeStruct((G, 1), jnp.float32),
    scratch_shapes=[
        pltpu.VMEM((G, D), jnp.float32),
        pltpu.VMEM((G, 1), jnp.float32),
    ],
)


def kernel(x, edge_index, edge_attr, batch, node_enc_w, node_enc_b,
           lin_w, lin_b, edge_w, edge_b, pred_w, pred_b):
    row = edge_index[0]
    col = edge_index[1]
    pad = EPAD - E
    padv = jnp.full((pad,), DUMMY, jnp.int32)
    row_pad = jnp.concatenate([row, padv])
    col_pad = jnp.concatenate([col, padv])
    rowp = row_pad.reshape(NTILES, NCH, CH)
    colp = col_pad.reshape(NTILES, NCH, CH)
    r3 = row_pad.reshape(NS, NCH2, CH)
    row_sc = jnp.stack([r3, r3 + NP])
    col_sc = col_pad.reshape(NS, NCH2, CH)
    ea_pad = jnp.concatenate(
        [edge_attr, jnp.zeros((pad, DE), jnp.float32)])
    zeros_nh = jnp.zeros((NP, DH), jnp.float32)

    degp = _deg_kernel()(rowp.reshape(NTILES, EPT))
    dinv2, ms0 = _tc_enc(degp, x, node_enc_w, node_enc_b[None, :],
                         lin_w[0])
    dinv1 = dinv2.reshape(NP)

    zeros_nq = jnp.zeros((NP, QW), jnp.float32)
    w = _wg_kernel()(rowp.reshape(NTILES, EPT), dinv1)
    q = _tc_q(w.reshape(EPAD, 1), ea_pad)
    s32 = _qs_kernel()(colp, q.reshape(NTILES, NCH, CH, QW), zeros_nq)
    agg = _agg_kernel()(row_sc, col_sc, ms0.reshape(NC * NP, DH), zeros_nh)

    ms = ms0
    for l in range(2):
        ms = _tc_layer(dinv2, ms, agg, s32, edge_w[l],
                       (edge_b[l] + lin_b[l])[None, :], lin_w[l + 1])
        agg = _agg_kernel()(row_sc, col_sc, ms.reshape(NC * NP, DH),
                            zeros_nh)

    batch_p = jnp.concatenate(
        [batch, jnp.full((NP - N,), G, jnp.int32)])[:, None]
    out = _tc_final(dinv2, ms, agg, s32, edge_w[2],
                    (edge_b[2] + lin_b[2])[None, :],
                    batch_p, pred_w, pred_b[:, None])
    return out


# speedup vs baseline: 8.3544x; 1.0509x over previous
import functools

import jax
import jax.numpy as jnp
from jax import lax
from jax.experimental import pallas as pl
from jax.experimental.pallas import tpu as pltpu
from jax.experimental.pallas import tpu_sc as plsc

N = 10000
D = 128
DE = 16
G = 16
NP = 10112
DUMMY = 10048
NC = 2
NS = 16
NTILES = NC * NS
E = 320000
EPT = 10240
CH = 128
NCH = EPT // CH
EPAD = EPT * NTILES
ROWS_PT = NP // NS
QW = 2 * DE

@functools.cache
def _mesh():
    return plsc.VectorSubcoreMesh(core_axis_name="c", subcore_axis_name="s",
                                  num_cores=NC, num_subcores=NS)


_SC_PARAMS = pltpu.CompilerParams(needs_layout_passes=False)


def _deg_body(row_hbm, out_hbm, rowbuf, hist):
    c = lax.axis_index("c")
    s = lax.axis_index("s")
    wid = s * NC + c
    pltpu.sync_copy(row_hbm.at[wid], rowbuf)
    zeros = jnp.zeros((16,), jnp.float32)

    def zbody(i, carry):
        for j in range(8):
            hist[pl.ds(i * 128 + j * 16, 16)] = zeros
        return carry

    lax.fori_loop(0, NP // 128, zbody, 0)
    ones = jnp.ones((16,), jnp.float32)

    def body(i, carry):
        for j in range(8):
            rv = rowbuf[pl.ds(i * 128 + j * 16, 16)]
            plsc.addupdate_scatter(hist, [rv], ones)
        return carry

    lax.fori_loop(0, EPT // 128, body, 0)
    pltpu.sync_copy(hist, out_hbm.at[wid])


@functools.cache
def _deg_kernel():
    return pl.kernel(
        _deg_body,
        out_type=jax.ShapeDtypeStruct((NTILES, NP), jnp.float32),
        mesh=_mesh(),
        compiler_params=_SC_PARAMS,
        scratch_types=[
            pltpu.VMEM((EPT,), jnp.int32),
            pltpu.VMEM((NP,), jnp.float32),
        ],
    )


DH = D // NC
EPT2 = EPAD // NS
NCH2 = EPT2 // CH


def _agg_body(row_hbm, col_hbm, ms_hbm, z_hbm, agg_out,
              row2d, col2d, gbuf0, gbuf1, gbuf2, gbuf3, agg_sp,
              sg0, sg1, sg2, sg3, ss0, ss1, ss2, ss3):
    c = lax.axis_index("c")
    s = lax.axis_index("s")
    r0 = s * ROWS_PT
    pltpu.sync_copy(z_hbm.at[pl.ds(r0, ROWS_PT)],
                    agg_sp.at[pl.ds(r0, ROWS_PT)])
    pltpu.sync_copy(row_hbm.at[c, s], row2d)
    pltpu.sync_copy(col_hbm.at[s], col2d)
    plsc.subcore_barrier()

    gbufs = (gbuf0, gbuf1, gbuf2, gbuf3)
    sgs = (sg0, sg1, sg2, sg3)
    sss = (ss0, ss1, ss2, ss3)
    P = 4

    def gather(k, b):
        pltpu.async_copy(ms_hbm.at[row2d.at[k]], gbufs[b], sgs[b])

    def wait_gather(k, b):
        pltpu.make_async_copy(ms_hbm.at[row2d.at[k]], gbufs[b],
                              sgs[b]).wait()

    def scatter(k, b):
        pltpu.async_copy(gbufs[b], agg_sp.at[col2d.at[k]], sss[b],
                         add=True)

    def wait_scatter(k, b):
        pltpu.make_async_copy(gbufs[b], agg_sp.at[col2d.at[k]],
                              sss[b]).wait()

    def sub(k, b):
        @pl.when(k >= P)
        def _():
            wait_scatter(k - P, b)

        gather(k, b)

        @pl.when(k >= P - 1)
        def _():
            wait_gather(k - (P - 1), (b + 1) % P)
            scatter(k - (P - 1), (b + 1) % P)

    def chunk(i, carry):
        for j in range(P):
            sub(P * i + j, j)
        return carry

    lax.fori_loop(0, NCH2 // P, chunk, 0)
    for k in range(NCH2 - (P - 1), NCH2):
        wait_gather(k, k % P)
        scatter(k, k % P)
    for k in range(NCH2 - P, NCH2):
        wait_scatter(k, k % P)
    plsc.subcore_barrier()
    pltpu.sync_copy(agg_sp.at[pl.ds(r0, ROWS_PT)],
                    agg_out.at[c, pl.ds(r0, ROWS_PT)])


@functools.cache
def _agg_kernel():
    return pl.kernel(
        _agg_body,
        out_type=jax.ShapeDtypeStruct((NC, NP, DH), jnp.float32),
        mesh=_mesh(),
        compiler_params=pltpu.CompilerParams(needs_layout_passes=False,
                                             use_tc_tiling_on_sc=False),
        scratch_types=[
            pltpu.VMEM((NCH2, CH), jnp.int32),
            pltpu.VMEM((NCH2, CH), jnp.int32),
            pltpu.VMEM((CH, DH), jnp.float32),
            pltpu.VMEM((CH, DH), jnp.float32),
            pltpu.VMEM((CH, DH), jnp.float32),
            pltpu.VMEM((CH, DH), jnp.float32),
            pltpu.VMEM_SHARED((NP, DH), jnp.float32),
        ] + [pltpu.SemaphoreType.DMA] * 8,
    )


def _wg_body(row_hbm, dinv_hbm, w_out, rowbuf, dinv_v, wbuf):
    c = lax.axis_index("c")
    s = lax.axis_index("s")
    wid = s * NC + c
    pltpu.sync_copy(row_hbm.at[wid], rowbuf)
    pltpu.sync_copy(dinv_hbm, dinv_v)

    def body(i, carry):
        for j in range(8):
            rv = rowbuf[pl.ds(i * 128 + j * 16, 16)]
            wbuf[pl.ds(i * 128 + j * 16, 16)] = plsc.load_gather(
                dinv_v, [rv])
        return carry

    lax.fori_loop(0, EPT // 128, body, 0)
    pltpu.sync_copy(wbuf, w_out.at[wid])


@functools.cache
def _wg_kernel():
    return pl.kernel(
        _wg_body,
        out_type=jax.ShapeDtypeStruct((NTILES, EPT), jnp.float32),
        mesh=_mesh(),
        compiler_params=_SC_PARAMS,
        scratch_types=[
            pltpu.VMEM((EPT,), jnp.int32),
            pltpu.VMEM((NP,), jnp.float32),
            pltpu.VMEM((EPT,), jnp.float32),
        ],
    )


def _tc_q_body(w_ref, ea_ref, q_ref):
    w = w_ref[...]
    q_ref[:, 0:DE] = w * ea_ref[...]
    q_ref[:, DE:DE + 1] = w
    q_ref[:, DE + 1:QW] = jnp.zeros((w.shape[0], QW - DE - 1), jnp.float32)


_BQ = EPAD // 40
_tc_q = pl.pallas_call(
    _tc_q_body,
    grid=(40,),
    in_specs=[
        pl.BlockSpec((_BQ, 1), lambda i: (i, 0)),
        pl.BlockSpec((_BQ, DE), lambda i: (i, 0)),
    ],
    out_specs=pl.BlockSpec((_BQ, QW), lambda i: (i, 0)),
    out_shape=jax.ShapeDtypeStruct((EPAD, QW), jnp.float32),
)


def _qs_body(col_hbm, q_hbm, zq_hbm, s_out, col2d, qbuf0, qbuf1, s_sp,
             sq0, sq1, ss0, ss1):
    c = lax.axis_index("c")
    s = lax.axis_index("s")
    wid = s * NC + c
    r0 = s * ROWS_PT
    pltpu.sync_copy(zq_hbm.at[pl.ds(r0, ROWS_PT)],
                    s_sp.at[pl.ds(r0, ROWS_PT)])
    pltpu.sync_copy(col_hbm.at[wid], col2d)
    plsc.subcore_barrier()

    qbufs = (qbuf0, qbuf1)
    sqs = (sq0, sq1)
    sss = (ss0, ss1)

    def stage(k, b):
        pltpu.async_copy(q_hbm.at[wid, k], qbufs[b], sqs[b])

    def wait_stage(k, b):
        pltpu.make_async_copy(q_hbm.at[wid, k], qbufs[b], sqs[b]).wait()

    def scatter(k, b):
        pltpu.async_copy(qbufs[b], s_sp.at[col2d.at[k]], sss[b], add=True)

    def wait_scatter(k, b):
        pltpu.make_async_copy(qbufs[b], s_sp.at[col2d.at[k]],
                              sss[b]).wait()

    def sub(k, b):
        @pl.when(k >= 2)
        def _():
            wait_scatter(k - 2, b)

        stage(k, b)

        @pl.when(k >= 1)
        def _():
            wait_stage(k - 1, 1 - b)
            scatter(k - 1, 1 - b)

    def chunk(i, carry):
        sub(2 * i, 0)
        sub(2 * i + 1, 1)
        return carry

    lax.fori_loop(0, NCH // 2, chunk, 0)
    wait_stage(NCH - 1, 1)
    scatter(NCH - 1, 1)
    wait_scatter(NCH - 2, 0)
    wait_scatter(NCH - 1, 1)
    plsc.subcore_barrier()
    pltpu.sync_copy(s_sp.at[pl.ds(r0, ROWS_PT)],
                    s_out.at[c, pl.ds(r0, ROWS_PT)])


@functools.cache
def _qs_kernel():
    return pl.kernel(
        _qs_body,
        out_type=jax.ShapeDtypeStruct((NC, NP, QW), jnp.float32),
        mesh=_mesh(),
        compiler_params=pltpu.CompilerParams(needs_layout_passes=False,
                                             use_tc_tiling_on_sc=False),
        scratch_types=[
            pltpu.VMEM((NCH, CH), jnp.int32),
            pltpu.VMEM((CH, QW), jnp.float32),
            pltpu.VMEM((CH, QW), jnp.float32),
            pltpu.VMEM_SHARED((NP, QW), jnp.float32),
        ] + [pltpu.SemaphoreType.DMA] * 4,
    )


def _tc_enc_body(degp_ref, x_ref, wenc_ref, benc_ref, w0_ref,
                 dinv_ref, ms0_ref):
    deg = jnp.sum(degp_ref[...], axis=0) + 1.0
    dinv = 1.0 / jnp.sqrt(deg)
    dinv_ref[...] = dinv[:, None]
    h0 = jnp.dot(x_ref[...], wenc_ref[...].T,
                 preferred_element_type=jnp.float32) + benc_ref[...]
    ms0 = jnp.dot(h0, w0_ref[...].T,
                  preferred_element_type=jnp.float32) * dinv[:N, None]
    zpad = jnp.zeros((NP - N, DH), jnp.float32)
    ms0_ref[0, 0:N, :] = ms0[:, 0:DH]
    ms0_ref[0, N:NP, :] = zpad
    ms0_ref[1, 0:N, :] = ms0[:, DH:D]
    ms0_ref[1, N:NP, :] = zpad


_tc_enc = pl.pallas_call(
    _tc_enc_body,
    out_shape=[
        jax.ShapeDtypeStruct((NP, 1), jnp.float32),
        jax.ShapeDtypeStruct((NC, NP, DH), jnp.float32),
    ],
)


def _tc_layer_body(dinv_ref, ms_ref, agg_ref, s32_ref, ew_ref, ebb_ref,
                   wn_ref, msn_ref, *, apply_relu):
    d = dinv_ref[...]
    d2 = d * d
    s16 = s32_ref[0, :, 0:DE] + s32_ref[1, :, 0:DE]
    ssum = s32_ref[0, :, DE:DE + 1] + s32_ref[1, :, DE:DE + 1]
    ew = ew_ref[...]
    t = (d * jnp.dot(s16, ew.T, preferred_element_type=jnp.float32)
         + d2 * ew[:, 0][None, :]
         + (d * ssum + d2) * ebb_ref[...])
    agg = jnp.concatenate([agg_ref[0], agg_ref[1]], axis=1)
    ms = jnp.concatenate([ms_ref[0], ms_ref[1]], axis=1)
    h = d * (agg + ms) + t
    if apply_relu:
        h = jnp.maximum(h, 0.0)
    msn = jnp.dot(h, wn_ref[...].T,
                  preferred_element_type=jnp.float32) * d
    msn_ref[0] = msn[:, 0:DH]
    msn_ref[1] = msn[:, DH:D]


_BN = NP // 8
_tc_layer = pl.pallas_call(
    functools.partial(_tc_layer_body, apply_relu=True),
    grid=(8,),
    in_specs=[
        pl.BlockSpec((_BN, 1), lambda i: (i, 0)),
        pl.BlockSpec((NC, _BN, DH), lambda i: (0, i, 0)),
        pl.BlockSpec((NC, _BN, DH), lambda i: (0, i, 0)),
        pl.BlockSpec((NC, _BN, QW), lambda i: (0, i, 0)),
        pl.BlockSpec((D, DE), lambda i: (0, 0)),
        pl.BlockSpec((1, D), lambda i: (0, 0)),
        pl.BlockSpec((D, D), lambda i: (0, 0)),
    ],
    out_specs=pl.BlockSpec((NC, _BN, DH), lambda i: (0, i, 0)),
    out_shape=jax.ShapeDtypeStruct((NC, NP, DH), jnp.float32),
)


def _tc_final_body(dinv_ref, ms_ref, agg_ref, s32_ref, ew_ref, ebb_ref,
                   batch_ref, pw_ref, pb_ref, out_ref, sums_ref, counts_ref):
    i = pl.program_id(0)
    d = dinv_ref[...]
    d2 = d * d
    s16 = s32_ref[0, :, 0:DE] + s32_ref[1, :, 0:DE]
    ssum = s32_ref[0, :, DE:DE + 1] + s32_ref[1, :, DE:DE + 1]
    ew = ew_ref[...]
    t = (d * jnp.dot(s16, ew.T, preferred_element_type=jnp.float32)
         + d2 * ew[:, 0][None, :]
         + (d * ssum + d2) * ebb_ref[...])
    agg = jnp.concatenate([agg_ref[0], agg_ref[1]], axis=1)
    ms = jnp.concatenate([ms_ref[0], ms_ref[1]], axis=1)
    h = d * (agg + ms) + t
    gid = lax.broadcasted_iota(jnp.int32, (_BN, G), 1)
    oh = jnp.where(batch_ref[...] == gid, 1.0, 0.0)
    part = lax.dot_general(oh, h, (((0,), (0,)), ((), ())),
                           preferred_element_type=jnp.float32)
    pc = jnp.sum(oh, axis=0)[:, None]

    @pl.when(i == 0)
    def _():
        sums_ref[...] = jnp.zeros((G, D), jnp.float32)
        counts_ref[...] = jnp.zeros((G, 1), jnp.float32)

    sums_ref[...] += part
    counts_ref[...] += pc

    @pl.when(i == 7)
    def _():
        pooled = sums_ref[...] / jnp.maximum(counts_ref[...], 1.0)
        out_ref[...] = (jnp.sum(pooled * pw_ref[...], axis=1, keepdims=True)
                        + pb_ref[0, 0])


_tc_final = pl.pallas_call(
    _tc_final_body,
    grid=(8,),
    in_specs=[
        pl.BlockSpec((_BN, 1), lambda i: (i, 0)),
        pl.BlockSpec((NC, _BN, DH), lambda i: (0, i, 0)),
        pl.BlockSpec((NC, _BN, DH), lambda i: (0, i, 0)),
        pl.BlockSpec((NC, _BN, QW), lambda i: (0, i, 0)),
        pl.BlockSpec((D, DE), lambda i: (0, 0)),
        pl.BlockSpec((1, D), lambda i: (0, 0)),
        pl.BlockSpec((_BN, 1), lambda i: (i, 0)),
        pl.BlockSpec((1, D), lambda i: (0, 0)),
        pl.BlockSpec((1, 1), lambda i: (0, 0)),
    ],
    out_specs=pl.BlockSpec((G, 1), lambda i: (0, 0)),
    out_shape=jax.ShapeDtypeStruct((G, 1), jnp.float32),
    scratch_shapes=[
        pltpu.VMEM((G, D), jnp.float32),
        pltpu.VMEM((G, 1), jnp.float32),
    ],
)


def kernel(x, edge_index, edge_attr, batch, node_enc_w, node_enc_b,
           lin_w, lin_b, edge_w, edge_b, pred_w, pred_b):
    row = edge_index[0]
    col = edge_index[1]
    pad = EPAD - E
    padv = jnp.full((pad,), DUMMY, jnp.int32)
    row_pad = jnp.concatenate([row, padv])
    col_pad = jnp.concatenate([col, padv])
    rowp = row_pad.reshape(NTILES, NCH, CH)
    colp = col_pad.reshape(NTILES, NCH, CH)
    r3 = row_pad.reshape(NS, NCH2, CH)
    row_sc = jnp.stack([r3, r3 + NP])
    col_sc = col_pad.reshape(NS, NCH2, CH)
    ea_pad = jnp.concatenate(
        [edge_attr, jnp.zeros((pad, DE), jnp.float32)])
    zeros_nh = jnp.zeros((NP, DH), jnp.float32)

    degp = _deg_kernel()(rowp.reshape(NTILES, EPT))
    dinv2, ms0 = _tc_enc(degp, x, node_enc_w, node_enc_b[None, :],
                         lin_w[0])
    dinv1 = dinv2.reshape(NP)

    zeros_nq = jnp.zeros((NP, QW), jnp.float32)
    w = _wg_kernel()(rowp.reshape(NTILES, EPT), dinv1)
    q = _tc_q(w.reshape(EPAD, 1), ea_pad)
    s32 = _qs_kernel()(colp, q.reshape(NTILES, NCH, CH, QW), zeros_nq)
    agg = _agg_kernel()(row_sc, col_sc, ms0.reshape(NC * NP, DH), zeros_nh)

    ms = ms0
    for l in range(2):
        ms = _tc_layer(dinv2, ms, agg, s32, edge_w[l],
                       (edge_b[l] + lin_b[l])[None, :], lin_w[l + 1])
        agg = _agg_kernel()(row_sc, col_sc, ms.reshape(NC * NP, DH),
                            zeros_nh)

    batch_p = jnp.concatenate(
        [batch, jnp.full((NP - N,), G, jnp.int32)])[:, None]
    out = _tc_final(dinv2, ms, agg, s32, edge_w[2],
                    (edge_b[2] + lin_b[2])[None, :],
                    batch_p, pred_w, pred_b[:, None])
    return out


# speedup vs baseline: 8.3870x; 1.0039x over previous
import functools

import jax
import jax.numpy as jnp
from jax import lax
from jax.experimental import pallas as pl
from jax.experimental.pallas import tpu as pltpu
from jax.experimental.pallas import tpu_sc as plsc

N = 10000
D = 128
DE = 16
G = 16
NP = 10112
DUMMY = 10048
NC = 2
NS = 16
NTILES = NC * NS
E = 320000
EPT = 10240
CH = 128
NCH = EPT // CH
EPAD = EPT * NTILES
ROWS_PT = NP // NS
QW = 2 * DE

@functools.cache
def _mesh():
    return plsc.VectorSubcoreMesh(core_axis_name="c", subcore_axis_name="s",
                                  num_cores=NC, num_subcores=NS)


_SC_PARAMS = pltpu.CompilerParams(needs_layout_passes=False)


def _deg_body(row_hbm, out_hbm, rowbuf, hist):
    c = lax.axis_index("c")
    s = lax.axis_index("s")
    wid = s * NC + c
    pltpu.sync_copy(row_hbm.at[wid], rowbuf)
    zeros = jnp.zeros((16,), jnp.float32)

    def zbody(i, carry):
        for j in range(8):
            hist[pl.ds(i * 128 + j * 16, 16)] = zeros
        return carry

    lax.fori_loop(0, NP // 128, zbody, 0)
    ones = jnp.ones((16,), jnp.float32)

    def body(i, carry):
        for j in range(8):
            rv = rowbuf[pl.ds(i * 128 + j * 16, 16)]
            plsc.addupdate_scatter(hist, [rv], ones)
        return carry

    lax.fori_loop(0, EPT // 128, body, 0)
    pltpu.sync_copy(hist, out_hbm.at[wid])


@functools.cache
def _deg_kernel():
    return pl.kernel(
        _deg_body,
        out_type=jax.ShapeDtypeStruct((NTILES, NP), jnp.float32),
        mesh=_mesh(),
        compiler_params=_SC_PARAMS,
        scratch_types=[
            pltpu.VMEM((EPT,), jnp.int32),
            pltpu.VMEM((NP,), jnp.float32),
        ],
    )


DH = D // NC
EPT2 = EPAD // NS
NCH2 = EPT2 // CH


def _agg_body(row_hbm, col_hbm, ms_hbm, z_hbm, agg_out,
              row2d, col2d, gbuf0, gbuf1, gbuf2, gbuf3, agg_sp,
              sg0, sg1, sg2, sg3, ss0, ss1, ss2, ss3):
    c = lax.axis_index("c")
    s = lax.axis_index("s")
    r0 = s * ROWS_PT
    pltpu.sync_copy(z_hbm.at[pl.ds(r0, ROWS_PT)],
                    agg_sp.at[pl.ds(r0, ROWS_PT)])
    pltpu.sync_copy(row_hbm.at[c, s], row2d)
    pltpu.sync_copy(col_hbm.at[s], col2d)
    plsc.subcore_barrier()

    gbufs = (gbuf0, gbuf1, gbuf2, gbuf3)
    sgs = (sg0, sg1, sg2, sg3)
    sss = (ss0, ss1, ss2, ss3)
    P = 4

    def gather(k, b):
        pltpu.async_copy(ms_hbm.at[row2d.at[k]], gbufs[b], sgs[b])

    def wait_gather(k, b):
        pltpu.make_async_copy(ms_hbm.at[row2d.at[k]], gbufs[b],
                              sgs[b]).wait()

    def scatter(k, b):
        pltpu.async_copy(gbufs[b], agg_sp.at[col2d.at[k]], sss[b],
                         add=True)

    def wait_scatter(k, b):
        pltpu.make_async_copy(gbufs[b], agg_sp.at[col2d.at[k]],
                              sss[b]).wait()

    def sub(k, b):
        @pl.when(k >= P)
        def _():
            wait_scatter(k - P, b)

        gather(k, b)

        @pl.when(k >= P - 1)
        def _():
            wait_gather(k - (P - 1), (b + 1) % P)
            scatter(k - (P - 1), (b + 1) % P)

    def chunk(i, carry):
        for j in range(P):
            sub(P * i + j, j)
        return carry

    lax.fori_loop(0, NCH2 // P, chunk, 0)
    for k in range(NCH2 - (P - 1), NCH2):
        wait_gather(k, k % P)
        scatter(k, k % P)
    for k in range(NCH2 - P, NCH2):
        wait_scatter(k, k % P)
    plsc.subcore_barrier()
    pltpu.sync_copy(agg_sp.at[pl.ds(r0, ROWS_PT)],
                    agg_out.at[c, pl.ds(r0, ROWS_PT)])


@functools.cache
def _agg_kernel():
    return pl.kernel(
        _agg_body,
        out_type=jax.ShapeDtypeStruct((NC, NP, DH), jnp.float32),
        mesh=_mesh(),
        compiler_params=pltpu.CompilerParams(needs_layout_passes=False,
                                             use_tc_tiling_on_sc=False),
        scratch_types=[
            pltpu.VMEM((NCH2, CH), jnp.int32),
            pltpu.VMEM((NCH2, CH), jnp.int32),
            pltpu.VMEM((CH, DH), jnp.float32),
            pltpu.VMEM((CH, DH), jnp.float32),
            pltpu.VMEM((CH, DH), jnp.float32),
            pltpu.VMEM((CH, DH), jnp.float32),
            pltpu.VMEM_SHARED((NP, DH), jnp.float32),
        ] + [pltpu.SemaphoreType.DMA] * 8,
    )


def _wg_body(row_hbm, dinv_hbm, w_out, rowbuf, dinv_v, wbuf):
    c = lax.axis_index("c")
    s = lax.axis_index("s")
    wid = s * NC + c
    pltpu.sync_copy(row_hbm.at[wid], rowbuf)
    pltpu.sync_copy(dinv_hbm, dinv_v)

    def body(i, carry):
        for j in range(8):
            rv = rowbuf[pl.ds(i * 128 + j * 16, 16)]
            wbuf[pl.ds(i * 128 + j * 16, 16)] = plsc.load_gather(
                dinv_v, [rv])
        return carry

    lax.fori_loop(0, EPT // 128, body, 0)
    pltpu.sync_copy(wbuf, w_out.at[wid])


@functools.cache
def _wg_kernel():
    return pl.kernel(
        _wg_body,
        out_type=jax.ShapeDtypeStruct((NTILES, EPT), jnp.float32),
        mesh=_mesh(),
        compiler_params=_SC_PARAMS,
        scratch_types=[
            pltpu.VMEM((EPT,), jnp.int32),
            pltpu.VMEM((NP,), jnp.float32),
            pltpu.VMEM((EPT,), jnp.float32),
        ],
    )


def _tc_q_body(w_ref, ea_ref, q_ref):
    w = w_ref[...]
    q_ref[:, 0:DE] = w * ea_ref[...]
    q_ref[:, DE:DE + 1] = w
    q_ref[:, DE + 1:QW] = jnp.zeros((w.shape[0], QW - DE - 1), jnp.float32)


_BQ = EPAD // 40
_tc_q = pl.pallas_call(
    _tc_q_body,
    grid=(40,),
    in_specs=[
        pl.BlockSpec((_BQ, 1), lambda i: (i, 0)),
        pl.BlockSpec((_BQ, DE), lambda i: (i, 0)),
    ],
    out_specs=pl.BlockSpec((_BQ, QW), lambda i: (i, 0)),
    out_shape=jax.ShapeDtypeStruct((EPAD, QW), jnp.float32),
)


def _qs_body(col_hbm, q_hbm, zq_hbm, s_out, col2d, qbuf0, qbuf1, s_sp,
             sq0, sq1, ss0, ss1):
    c = lax.axis_index("c")
    s = lax.axis_index("s")
    wid = s * NC + c
    r0 = s * ROWS_PT
    pltpu.sync_copy(zq_hbm.at[pl.ds(r0, ROWS_PT)],
                    s_sp.at[pl.ds(r0, ROWS_PT)])
    pltpu.sync_copy(col_hbm.at[wid], col2d)
    plsc.subcore_barrier()

    qbufs = (qbuf0, qbuf1)
    sqs = (sq0, sq1)
    sss = (ss0, ss1)

    def stage(k, b):
        pltpu.async_copy(q_hbm.at[wid, k], qbufs[b], sqs[b])

    def wait_stage(k, b):
        pltpu.make_async_copy(q_hbm.at[wid, k], qbufs[b], sqs[b]).wait()

    def scatter(k, b):
        pltpu.async_copy(qbufs[b], s_sp.at[col2d.at[k]], sss[b], add=True)

    def wait_scatter(k, b):
        pltpu.make_async_copy(qbufs[b], s_sp.at[col2d.at[k]],
                              sss[b]).wait()

    def sub(k, b):
        @pl.when(k >= 2)
        def _():
            wait_scatter(k - 2, b)

        stage(k, b)

        @pl.when(k >= 1)
        def _():
            wait_stage(k - 1, 1 - b)
            scatter(k - 1, 1 - b)

    def chunk(i, carry):
        sub(2 * i, 0)
        sub(2 * i + 1, 1)
        return carry

    lax.fori_loop(0, NCH // 2, chunk, 0)
    wait_stage(NCH - 1, 1)
    scatter(NCH - 1, 1)
    wait_scatter(NCH - 2, 0)
    wait_scatter(NCH - 1, 1)
    plsc.subcore_barrier()
    pltpu.sync_copy(s_sp.at[pl.ds(r0, ROWS_PT)],
                    s_out.at[c, pl.ds(r0, ROWS_PT)])


def _st2_body(row_hbm, col_hbm, dinv_hbm, ea_hbm, zq_hbm, s_out,
              rowflat, col2d, dinv_v, eabuf0, eabuf1, qbuf0, qbuf1, s_sp,
              se0, se1, ss0, ss1):
    c = lax.axis_index("c")
    s = lax.axis_index("s")
    wid = s * NC + c
    r0 = s * ROWS_PT
    pltpu.sync_copy(zq_hbm.at[pl.ds(r0, ROWS_PT)],
                    s_sp.at[pl.ds(r0, ROWS_PT)])
    pltpu.sync_copy(zq_hbm.at[pl.ds(0, CH)], qbuf0)
    pltpu.sync_copy(zq_hbm.at[pl.ds(0, CH)], qbuf1)
    pltpu.sync_copy(row_hbm.at[wid], rowflat)
    pltpu.sync_copy(col_hbm.at[wid], col2d)
    pltpu.sync_copy(dinv_hbm, dinv_v)
    plsc.subcore_barrier()

    eabufs = (eabuf0, eabuf1)
    qbufs = (qbuf0, qbuf1)
    ses = (se0, se1)
    sss = (ss0, ss1)
    iota = lax.iota(jnp.int32, 16)
    fde = jnp.full((16,), DE, jnp.int32)

    def stage(k, b):
        pltpu.async_copy(ea_hbm.at[wid, k], eabufs[b], ses[b])

    def wait_stage(k, b):
        pltpu.make_async_copy(ea_hbm.at[wid, k], eabufs[b], ses[b]).wait()

    def scatter(k, b):
        pltpu.async_copy(qbufs[b], s_sp.at[col2d.at[k]], sss[b], add=True)

    def wait_scatter(k, b):
        pltpu.make_async_copy(qbufs[b], s_sp.at[col2d.at[k]],
                              sss[b]).wait()

    def build(k, b):
        for g in range(CH // 16):
            ids = iota + (g * 16)
            rv = rowflat[pl.ds(k * CH + g * 16, 16)]
            dv = plsc.load_gather(dinv_v, [rv])
            plsc.store_scatter(qbufs[b], [ids, fde], dv)
            for f in range(DE):
                fv = jnp.full((16,), f, jnp.int32)
                ev = plsc.load_gather(eabufs[b], [ids, fv])
                plsc.store_scatter(qbufs[b], [ids, fv], ev * dv)

    def sub(k, b):
        @pl.when(k >= 2)
        def _():
            wait_scatter(k - 2, b)

        wait_stage(k, b)
        build(k, b)
        scatter(k, b)

        @pl.when(k + 2 < NCH)
        def _():
            stage(k + 2, b)

    stage(0, 0)
    stage(1, 1)

    def chunk(i, carry):
        sub(2 * i, 0)
        sub(2 * i + 1, 1)
        return carry

    lax.fori_loop(0, NCH // 2, chunk, 0)
    wait_scatter(NCH - 2, 0)
    wait_scatter(NCH - 1, 1)
    plsc.subcore_barrier()
    pltpu.sync_copy(s_sp.at[pl.ds(r0, ROWS_PT)],
                    s_out.at[c, pl.ds(r0, ROWS_PT)])


@functools.cache
def _st2_kernel():
    return pl.kernel(
        _st2_body,
        out_type=jax.ShapeDtypeStruct((NC, NP, QW), jnp.float32),
        mesh=_mesh(),
        compiler_params=pltpu.CompilerParams(needs_layout_passes=False,
                                             use_tc_tiling_on_sc=False),
        scratch_types=[
            pltpu.VMEM((EPT,), jnp.int32),
            pltpu.VMEM((NCH, CH), jnp.int32),
            pltpu.VMEM((NP,), jnp.float32),
            pltpu.VMEM((CH, DE), jnp.float32),
            pltpu.VMEM((CH, DE), jnp.float32),
            pltpu.VMEM((CH, QW), jnp.float32),
            pltpu.VMEM((CH, QW), jnp.float32),
            pltpu.VMEM_SHARED((NP, QW), jnp.float32),
        ] + [pltpu.SemaphoreType.DMA] * 4,
    )


@functools.cache
def _qs_kernel():
    return pl.kernel(
        _qs_body,
        out_type=jax.ShapeDtypeStruct((NC, NP, QW), jnp.float32),
        mesh=_mesh(),
        compiler_params=pltpu.CompilerParams(needs_layout_passes=False,
                                             use_tc_tiling_on_sc=False),
        scratch_types=[
            pltpu.VMEM((NCH, CH), jnp.int32),
            pltpu.VMEM((CH, QW), jnp.float32),
            pltpu.VMEM((CH, QW), jnp.float32),
            pltpu.VMEM_SHARED((NP, QW), jnp.float32),
        ] + [pltpu.SemaphoreType.DMA] * 4,
    )


def _tc_enc_body(degp_ref, x_ref, wenc_ref, benc_ref, w0_ref,
                 dinv_ref, ms0_ref):
    deg = jnp.sum(degp_ref[...], axis=0) + 1.0
    dinv = 1.0 / jnp.sqrt(deg)
    dinv_ref[...] = dinv[:, None]
    h0 = jnp.dot(x_ref[...], wenc_ref[...].T,
                 preferred_element_type=jnp.float32) + benc_ref[...]
    ms0 = jnp.dot(h0, w0_ref[...].T,
                  preferred_element_type=jnp.float32) * dinv[:N, None]
    zpad = jnp.zeros((NP - N, DH), jnp.float32)
    ms0_ref[0, 0:N, :] = ms0[:, 0:DH]
    ms0_ref[0, N:NP, :] = zpad
    ms0_ref[1, 0:N, :] = ms0[:, DH:D]
    ms0_ref[1, N:NP, :] = zpad


_tc_enc = pl.pallas_call(
    _tc_enc_body,
    out_shape=[
        jax.ShapeDtypeStruct((NP, 1), jnp.float32),
        jax.ShapeDtypeStruct((NC, NP, DH), jnp.float32),
    ],
)


def _tc_layer_body(dinv_ref, ms_ref, agg_ref, s32_ref, ew_ref, ebb_ref,
                   wn_ref, msn_ref, *, apply_relu):
    d = dinv_ref[...]
    d2 = d * d
    s16 = s32_ref[0, :, 0:DE] + s32_ref[1, :, 0:DE]
    ssum = s32_ref[0, :, DE:DE + 1] + s32_ref[1, :, DE:DE + 1]
    ew = ew_ref[...]
    t = (d * jnp.dot(s16, ew.T, preferred_element_type=jnp.float32)
         + d2 * ew[:, 0][None, :]
         + (d * ssum + d2) * ebb_ref[...])
    agg = jnp.concatenate([agg_ref[0], agg_ref[1]], axis=1)
    ms = jnp.concatenate([ms_ref[0], ms_ref[1]], axis=1)
    h = d * (agg + ms) + t
    if apply_relu:
        h = jnp.maximum(h, 0.0)
    msn = jnp.dot(h, wn_ref[...].T,
                  preferred_element_type=jnp.float32) * d
    msn_ref[0] = msn[:, 0:DH]
    msn_ref[1] = msn[:, DH:D]


_BN = NP // 8
_tc_layer = pl.pallas_call(
    functools.partial(_tc_layer_body, apply_relu=True),
    grid=(8,),
    in_specs=[
        pl.BlockSpec((_BN, 1), lambda i: (i, 0)),
        pl.BlockSpec((NC, _BN, DH), lambda i: (0, i, 0)),
        pl.BlockSpec((NC, _BN, DH), lambda i: (0, i, 0)),
        pl.BlockSpec((NC, _BN, QW), lambda i: (0, i, 0)),
        pl.BlockSpec((D, DE), lambda i: (0, 0)),
        pl.BlockSpec((1, D), lambda i: (0, 0)),
        pl.BlockSpec((D, D), lambda i: (0, 0)),
    ],
    out_specs=pl.BlockSpec((NC, _BN, DH), lambda i: (0, i, 0)),
    out_shape=jax.ShapeDtypeStruct((NC, NP, DH), jnp.float32),
)


def _tc_final_body(dinv_ref, ms_ref, agg_ref, s32_ref, ew_ref, ebb_ref,
                   batch_ref, pw_ref, pb_ref, out_ref, sums_ref, counts_ref):
    i = pl.program_id(0)
    d = dinv_ref[...]
    d2 = d * d
    s16 = s32_ref[0, :, 0:DE] + s32_ref[1, :, 0:DE]
    ssum = s32_ref[0, :, DE:DE + 1] + s32_ref[1, :, DE:DE + 1]
    ew = ew_ref[...]
    t = (d * jnp.dot(s16, ew.T, preferred_element_type=jnp.float32)
         + d2 * ew[:, 0][None, :]
         + (d * ssum + d2) * ebb_ref[...])
    agg = jnp.concatenate([agg_ref[0], agg_ref[1]], axis=1)
    ms = jnp.concatenate([ms_ref[0], ms_ref[1]], axis=1)
    h = d * (agg + ms) + t
    gid = lax.broadcasted_iota(jnp.int32, (_BN, G), 1)
    oh = jnp.where(batch_ref[...] == gid, 1.0, 0.0)
    part = lax.dot_general(oh, h, (((0,), (0,)), ((), ())),
                           preferred_element_type=jnp.float32)
    pc = jnp.sum(oh, axis=0)[:, None]

    @pl.when(i == 0)
    def _():
        sums_ref[...] = jnp.zeros((G, D), jnp.float32)
        counts_ref[...] = jnp.zeros((G, 1), jnp.float32)

    sums_ref[...] += part
    counts_ref[...] += pc

    @pl.when(i == 7)
    def _():
        pooled = sums_ref[...] / jnp.maximum(counts_ref[...], 1.0)
        out_ref[...] = (jnp.sum(pooled * pw_ref[...], axis=1, keepdims=True)
                        + pb_ref[0, 0])


_tc_final = pl.pallas_call(
    _tc_final_body,
    grid=(8,),
    in_specs=[
        pl.BlockSpec((_BN, 1), lambda i: (i, 0)),
        pl.BlockSpec((NC, _BN, DH), lambda i: (0, i, 0)),
        pl.BlockSpec((NC, _BN, DH), lambda i: (0, i, 0)),
        pl.BlockSpec((NC, _BN, QW), lambda i: (0, i, 0)),
        pl.BlockSpec((D, DE), lambda i: (0, 0)),
        pl.BlockSpec((1, D), lambda i: (0, 0)),
        pl.BlockSpec((_BN, 1), lambda i: (i, 0)),
        pl.BlockSpec((1, D), lambda i: (0, 0)),
        pl.BlockSpec((1, 1), lambda i: (0, 0)),
    ],
    out_specs=pl.BlockSpec((G, 1), lambda i: (0, 0)),
    out_shape=jax.ShapeDtypeStruct((G, 1), jnp.float32),
    scratch_shapes=[
        pltpu.VMEM((G, D), jnp.float32),
        pltpu.VMEM((G, 1), jnp.float32),
    ],
)


def kernel(x, edge_index, edge_attr, batch, node_enc_w, node_enc_b,
           lin_w, lin_b, edge_w, edge_b, pred_w, pred_b):
    row = edge_index[0]
    col = edge_index[1]
    pad = EPAD - E
    padv = jnp.full((pad,), DUMMY, jnp.int32)
    row_pad = jnp.concatenate([row, padv])
    col_pad = jnp.concatenate([col, padv])
    rowp = row_pad.reshape(NTILES, NCH, CH)
    colp = col_pad.reshape(NTILES, NCH, CH)
    r3 = row_pad.reshape(NS, NCH2, CH)
    row_sc = jnp.stack([r3, r3 + NP])
    col_sc = col_pad.reshape(NS, NCH2, CH)
    ea_pad = jnp.concatenate(
        [edge_attr, jnp.zeros((pad, DE), jnp.float32)])
    zeros_nh = jnp.zeros((NP, DH), jnp.float32)

    degp = _deg_kernel()(rowp.reshape(NTILES, EPT))
    dinv2, ms0 = _tc_enc(degp, x, node_enc_w, node_enc_b[None, :],
                         lin_w[0])
    dinv1 = dinv2.reshape(NP)

    zeros_nq = jnp.zeros((NP, QW), jnp.float32)
    s32 = _st2_kernel()(rowp.reshape(NTILES, EPT), colp, dinv1,
                        ea_pad.reshape(NTILES, NCH, CH, DE), zeros_nq)
    agg = _agg_kernel()(row_sc, col_sc, ms0.reshape(NC * NP, DH), zeros_nh)

    ms = ms0
    for l in range(2):
        ms = _tc_layer(dinv2, ms, agg, s32, edge_w[l],
                       (edge_b[l] + lin_b[l])[None, :], lin_w[l + 1])
        agg = _agg_kernel()(row_sc, col_sc, ms.reshape(NC * NP, DH),
                            zeros_nh)

    batch_p = jnp.concatenate(
        [batch, jnp.full((NP - N,), G, jnp.int32)])[:, None]
    out = _tc_final(dinv2, ms, agg, s32, edge_w[2],
                    (edge_b[2] + lin_b[2])[None, :],
                    batch_p, pred_w, pred_b[:, None])
    return out
